# baseline (speedup 1.0000x reference)
import jax
import jax.numpy as jnp
from jax.experimental import pallas as pl
from jax.experimental.pallas import tpu as pltpu

_B = 128
_N = 128
_E = _N * (_N - 1)
_IB = 128
_NI = _N // _IB
_RP = _N - 1


def _edge_kernel(xT_ref, xs_ref, ea_ref, ep_ref, rel_ref, rinv_ref, rows_ref):
    ib = pl.program_id(1)
    i0 = ib * _IB

    @pl.when(ib == 0)
    def _():
        xT = xT_ref[0]
        px = xT[0:1]
        py = xT[1:2]
        vx = xT[2:3]
        vy = xT[3:4]
        ex = xT[4:5]
        ey = xT[5:6]
        theta = jnp.arctan2(vy, vx)
        c = jnp.cos(theta)
        s = jnp.sin(theta)

        vl0 = c * vx + s * vy
        vl1 = -s * vx + c * vy
        el0 = c * ex + s * ey
        el1 = -s * ex + c * ey
        o_dxl = c * (-px) + s * (-py)
        o_dyl = -s * (-px) + c * (-py)
        o_drot = jnp.arctan2(jnp.sin(-theta), jnp.cos(-theta))
        o_r = jnp.sqrt(o_dxl * o_dxl + o_dyl * o_dyl + 1e-12)
        o_ang = jnp.arctan2(o_dyl, o_dxl)
        o_dvxl = c * (1.0 - vx) + s * (0.0 - vy)
        o_dvyl = -s * (1.0 - vx) + c * (0.0 - vy)
        rows24 = jnp.concatenate(
            [px, py, vx, vy, theta, c, s,
             vl0, vl1, el0, el1, o_dxl, o_dyl, o_drot, o_r, o_ang,
             o_dvxl, o_dvyl,
             c, -s, s, c, jnp.zeros((2, _N), jnp.float32)],
            axis=0)
        rows_ref[...] = rows24

        t16 = rows24[7:23].T
        rel_ref[0] = t16[:, 0:11]
        rinv_ref[0] = t16[:, 11:15].reshape(_N, 2, 2)

    rows = rows_ref[...]
    px = rows[0:1]
    py = rows[1:2]
    vx = rows[2:3]
    vy = rows[3:4]
    theta = rows[4:5]
    c = rows[5:6]
    s = rows[6:7]
    rel_rows = rows[7:18]

    xs = xs_ref[0]
    pxi = xs[:, 0:1]
    pyi = xs[:, 1:2]
    vxi = xs[:, 2:3]
    vyi = xs[:, 3:4]
    thetai = jnp.arctan2(vyi, vxi)

    dx = pxi - px
    dy = pyi - py
    dxl = c * dx + s * dy
    dyl = -s * dx + c * dy
    dr = thetai - theta
    drot = jnp.arctan2(jnp.sin(dr), jnp.cos(dr))
    r = jnp.sqrt(dxl * dxl + dyl * dyl + 1e-12)
    ang = jnp.arctan2(dyl, dxl)
    dvx = vxi - vx
    dvy = vyi - vy
    dvxl = c * dvx + s * dvy
    dvyl = -s * dvx + c * dvy

    jj = jax.lax.broadcasted_iota(jnp.int32, (_IB, _N), 1)
    ii = i0 + jax.lax.broadcasted_iota(jnp.int32, (_IB, _N), 0)
    cond = jj < ii

    def drop(m):
        m_shift = jnp.concatenate([m[:, 1:], m[:, -1:]], axis=1)
        return jnp.where(cond, m, m_shift)

    maps_d = [drop(m) for m in (dxl, dyl, drot, r, ang, dvxl, dvyl)]

    jj11 = jax.lax.broadcasted_iota(jnp.int32, (11, _N), 1)
    rel_shift = jnp.concatenate([rel_rows[:, 1:], rel_rows[:, -1:]], axis=1)
    zeros6 = jnp.zeros((6, _N), jnp.float32)

    zeros14 = jnp.zeros((14, _N), jnp.float32)
    for g in range(_IB // 8):
        pieces = []
        for u in range(8):
            i = g * 8 + u
            reld = jnp.where(jj11 < i0 + i, rel_rows, rel_shift)
            pieces += [m[i:i + 1] for m in maps_d] + [reld, zeros14]
        t = jnp.concatenate(pieces, axis=0).T
        for u in range(8):
            i = g * 8 + u
            ea_ref[0, i, :, :] = t[:_RP, 32 * u:32 * u + 18]
            ep_ref[0, i, :, :] = t[:_RP, 32 * u + 2:32 * u + 5]


def kernel(x):
    xT = jnp.transpose(x, (0, 2, 1))
    out_shapes = (
        jax.ShapeDtypeStruct((_B, _N, _RP, 18), jnp.float32),
        jax.ShapeDtypeStruct((_B, _N, _RP, 3), jnp.float32),
        jax.ShapeDtypeStruct((_B, _N, 11), jnp.float32),
        jax.ShapeDtypeStruct((_B, _N, 2, 2), jnp.float32),
    )
    ea, ep, rel, rinv = pl.pallas_call(
        _edge_kernel,
        grid=(_B, _NI),
        in_specs=[
            pl.BlockSpec((1, 6, _N), lambda b, i: (b, 0, 0)),
            pl.BlockSpec((1, _IB, 6), lambda b, i: (b, i, 0)),
        ],
        out_specs=[
            pl.BlockSpec((1, _IB, _RP, 18), lambda b, i: (b, i, 0, 0)),
            pl.BlockSpec((1, _IB, _RP, 3), lambda b, i: (b, i, 0, 0)),
            pl.BlockSpec((1, _N, 11), lambda b, i: (b, 0, 0)),
            pl.BlockSpec((1, _N, 2, 2), lambda b, i: (b, 0, 0, 0)),
        ],
        out_shape=out_shapes,
        scratch_shapes=[pltpu.VMEM((24, _N), jnp.float32)],
    )(xT, x)
    return rel, rinv, ea.reshape(_B, _E, 18), ep.reshape(_B, _E, 3)
